# baseline (speedup 1.0000x reference)
import functools

import jax
import jax.numpy as jnp
from jax import lax
from jax.experimental import pallas as pl
from jax.experimental.pallas import tpu as pltpu
from jax.experimental.pallas import tpu_sc as plsc

N = 10000
E = 320000
D = 128
H = 128

NW = 32
R = 320
NPAD = NW * R
CH = 2000
NCHUNK = E // CH
GB = 128
TRASH = ((CH + GB - 1) // GB) * GB
SELCAP = TRASH + 16
NEG = float("-inf")
LANES = 16


def _sc_agg_body(xsrc_hbm, src_hbm, dst_hbm, amax_hbm, asum_hbm,
                 dst_buf, src_buf, sel_dst, sel_src, rows_buf,
                 amax_acc, asum_acc):
    cid = lax.axis_index("c")
    sid = lax.axis_index("s")
    wid = sid * 2 + cid
    lo = wid * R

    def _init_acc(r, _):
        for f in range(D // LANES):
            s = pl.ds(f * LANES, LANES)
            amax_acc[r, s] = jnp.full((LANES,), NEG, jnp.float32)
            asum_acc[r, s] = jnp.zeros((LANES,), jnp.float32)
        return 0
    lax.fori_loop(0, R + 1, _init_acc, 0)

    def _init_sel(i, _):
        sel_src[pl.ds(i * LANES, LANES)] = jnp.zeros((LANES,), jnp.int32)
        return 0
    lax.fori_loop(0, SELCAP // LANES, _init_sel, 0)

    def _chunk(c, _):
        base = c * CH
        pltpu.sync_copy(dst_hbm.at[pl.ds(base, CH)], dst_buf)
        pltpu.sync_copy(src_hbm.at[pl.ds(base, CH)], src_buf)

        lane = lax.iota(jnp.int32, LANES)

        def _select(i, cnt):
            s = pl.ds(i * LANES, LANES)
            dv = dst_buf[s]
            sv = src_buf[s]
            m = (dv >= lo) & (dv < lo + R)
            cs = plsc.cumsum(m.astype(jnp.int32))
            p = jnp.where(m, cnt + cs - 1, TRASH + lane)
            plsc.store_scatter(sel_dst, [p], dv - lo)
            plsc.store_scatter(sel_src, [p], sv)
            return cnt + cs[LANES - 1]
        k = lax.fori_loop(0, CH // LANES, _select, jnp.int32(0))

        sel_dst[pl.ds(k, LANES)] = jnp.full((LANES,), R, jnp.int32)
        sel_src[pl.ds(k, LANES)] = jnp.zeros((LANES,), jnp.int32)
        kpad = ((k + LANES - 1) // LANES) * LANES

        def _batch(g, _):
            off = g * GB
            pltpu.sync_copy(xsrc_hbm.at[sel_src.at[pl.ds(off, GB)]],
                            rows_buf)

            def _evec(eb, _):
                dv = sel_dst[pl.ds(off + eb * LANES, LANES)]
                for j in range(LANES):
                    dloc = dv[j]
                    e = eb * LANES + j
                    for f in range(D // LANES):
                        s = pl.ds(f * LANES, LANES)
                        rv = rows_buf[e, s]
                        amax_acc[dloc, s] = jnp.maximum(amax_acc[dloc, s],
                                                        rv)
                        asum_acc[dloc, s] = asum_acc[dloc, s] + rv
                return 0
            lax.fori_loop(0, jnp.minimum(GB, kpad - off) // LANES, _evec, 0)
            return 0
        lax.fori_loop(0, (kpad + GB - 1) // GB, _batch, 0)
        return 0
    lax.fori_loop(0, NCHUNK, _chunk, 0)

    def _fin(r, _):
        for f in range(D // LANES):
            s = pl.ds(f * LANES, LANES)
            v = amax_acc[r, s]
            amax_acc[r, s] = jnp.where(v > NEG, v, jnp.float32(0.0))
        return 0
    lax.fori_loop(0, R, _fin, 0)

    pltpu.sync_copy(amax_acc.at[pl.ds(0, R)], amax_hbm.at[pl.ds(lo, R)])
    pltpu.sync_copy(asum_acc.at[pl.ds(0, R)], asum_hbm.at[pl.ds(lo, R)])


def _sc_aggregate(x_src, src, dst):
    mesh = plsc.VectorSubcoreMesh(core_axis_name="c", subcore_axis_name="s")
    f = pl.kernel(
        _sc_agg_body,
        out_type=[
            jax.ShapeDtypeStruct((NPAD, D), jnp.float32),
            jax.ShapeDtypeStruct((NPAD, D), jnp.float32),
        ],
        mesh=mesh,
        compiler_params=pltpu.CompilerParams(needs_layout_passes=False),
        scratch_types=[
            pltpu.VMEM((CH,), jnp.int32),
            pltpu.VMEM((CH,), jnp.int32),
            pltpu.VMEM((SELCAP,), jnp.int32),
            pltpu.VMEM((SELCAP,), jnp.int32),
            pltpu.VMEM((GB, D), jnp.float32),
            pltpu.VMEM((R + 1, D), jnp.float32),
            pltpu.VMEM((R + 1, D), jnp.float32),
        ],
    )
    return f(x_src, src, dst)


BLK = 1000


def _tc_mlp_body(amax_ref, asum_ref, xdst_ref, w1_ref, b1_ref, w2_ref,
                 b2_ref, w3_ref, b3_ref, out_ref):
    w2 = w2_ref[...]
    w2m = w2[:, :D] + w2[:, D:2 * D]
    w2s = w2[:, 2 * D:]
    w3 = w3_ref[...]
    dn = (((1,), (1,)), ((), ()))
    h2 = (lax.dot_general(amax_ref[...], w2m, dn,
                          preferred_element_type=jnp.float32)
          + lax.dot_general(asum_ref[...], w2s, dn,
                            preferred_element_type=jnp.float32)
          + b2_ref[...])
    h1 = (lax.dot_general(xdst_ref[...], w1_ref[...], dn,
                          preferred_element_type=jnp.float32)
          + b1_ref[...])
    h2 = jnp.maximum(h2, 0.0)
    h1 = jnp.maximum(h1, 0.0)
    o = (lax.dot_general(h2, w3[:, :H], dn,
                         preferred_element_type=jnp.float32)
         + lax.dot_general(h1, w3[:, H:], dn,
                           preferred_element_type=jnp.float32)
         + b3_ref[...])
    out_ref[...] = jnp.maximum(o, 0.0)


def _tc_mlp(amax, asum, x_dst, fc1_w, fc1_b, fc2_w, fc2_b, fc3_w, fc3_b):
    grid = (N // BLK,)
    row_spec = pl.BlockSpec((BLK, D), lambda i: (i, 0))
    full = lambda shape: pl.BlockSpec(shape, lambda i: (0, 0))
    return pl.pallas_call(
        _tc_mlp_body,
        grid=grid,
        in_specs=[
            row_spec, row_spec, row_spec,
            full((H, D)), full((1, H)),
            full((H, 3 * D)), full((1, H)),
            full((H, 2 * H)), full((1, H)),
        ],
        out_specs=pl.BlockSpec((BLK, H), lambda i: (i, 0)),
        out_shape=jax.ShapeDtypeStruct((N, H), jnp.float32),
    )(amax, asum, x_dst, fc1_w, fc1_b.reshape(1, H), fc2_w,
      fc2_b.reshape(1, H), fc3_w, fc3_b.reshape(1, H))


@jax.jit
def kernel(x_src, x_dst, edge_index, fc1_w, fc1_b, fc2_w, fc2_b,
           fc3_w, fc3_b):
    src = edge_index[0]
    dst = edge_index[1]
    amax, asum = _sc_aggregate(x_src, src, dst)
    return _tc_mlp(amax, asum, x_dst, fc1_w, fc1_b, fc2_w, fc2_b,
                   fc3_w, fc3_b)


# speedup vs baseline: 4.1419x; 4.1419x over previous
import jax
import jax.numpy as jnp
from jax import lax
from jax.experimental import pallas as pl
from jax.experimental.pallas import tpu as pltpu
from jax.experimental.pallas import tpu_sc as plsc

N = 10000
E = 320000
D = 128
H = 128

NW = 32
FPT = D // NW
CH = 2000
NCHUNK = E // CH
NEG = float("-inf")
LANES = 16


def _sc_agg_body(xsrcT_hbm, src_hbm, dst_hbm, amaxT_hbm, asumT_hbm,
                 dst_buf, src_buf, xslice, amax, asum):
    wid = lax.axis_index("s") * 2 + lax.axis_index("c")
    base = wid * (FPT * N)

    pltpu.sync_copy(xsrcT_hbm.at[pl.ds(base, FPT * N)], xslice)

    ninf = jnp.full((LANES,), NEG, jnp.float32)
    zero = jnp.zeros((LANES,), jnp.float32)

    def _init(i, _):
        s = pl.ds(i * LANES, LANES)
        amax[s] = ninf
        asum[s] = zero
        return 0
    lax.fori_loop(0, (FPT * N) // LANES, _init, 0)

    def _chunk(c, _):
        eb = c * CH
        pltpu.sync_copy(dst_hbm.at[pl.ds(eb, CH)], dst_buf)
        pltpu.sync_copy(src_hbm.at[pl.ds(eb, CH)], src_buf)

        lane = lax.iota(jnp.int32, LANES)

        def _vreg(i, _):
            s = pl.ds(i * LANES, LANES)
            dv = dst_buf[s]
            sv = src_buf[s]
            for c in range(FPT):
                gidx = sv + c * N
                aidx = dv + c * N
                v = plsc.load_gather(xslice, [gidx])
                plsc.addupdate_scatter(asum, [aidx], v)
                a = plsc.load_gather(amax, [aidx])
                plsc.store_scatter(amax, [aidx], jnp.maximum(a, v))

                def _body(carry):
                    a2 = plsc.load_gather(amax, [aidx])
                    need = a2 < v
                    eidx = jnp.where(need, aidx, FPT * N + lane)
                    plsc.store_scatter(amax, [eidx], jnp.maximum(a2, v))
                    return (jnp.any(need), 0)
                lax.while_loop(lambda carry: carry[0], _body, (True, 0))
            return 0
        lax.fori_loop(0, CH // LANES, _vreg, 0)
        return 0
    lax.fori_loop(0, NCHUNK, _chunk, 0)

    def _fin(i, _):
        s = pl.ds(i * LANES, LANES)
        v = amax[s]
        amax[s] = jnp.where(v > NEG, v, 0.0)
        return 0
    lax.fori_loop(0, (FPT * N) // LANES, _fin, 0)

    pltpu.sync_copy(amax.at[pl.ds(0, FPT * N)],
                    amaxT_hbm.at[pl.ds(base, FPT * N)])
    pltpu.sync_copy(asum, asumT_hbm.at[pl.ds(base, FPT * N)])


def _sc_aggregate(x_srcT_flat, src, dst):
    mesh = plsc.VectorSubcoreMesh(core_axis_name="c", subcore_axis_name="s")
    f = pl.kernel(
        _sc_agg_body,
        out_type=[
            jax.ShapeDtypeStruct((D * N,), jnp.float32),
            jax.ShapeDtypeStruct((D * N,), jnp.float32),
        ],
        mesh=mesh,
        compiler_params=pltpu.CompilerParams(needs_layout_passes=False),
        scratch_types=[
            pltpu.VMEM((CH,), jnp.int32),
            pltpu.VMEM((CH,), jnp.int32),
            pltpu.VMEM((FPT * N,), jnp.float32),
            pltpu.VMEM((FPT * N + LANES,), jnp.float32),
            pltpu.VMEM((FPT * N,), jnp.float32),
        ],
    )
    return f(x_srcT_flat, src, dst)


def _tc_mlp_body(amaxT_ref, asumT_ref, xdst_ref, w1_ref, b1_ref, w2_ref,
                 b2_ref, w3_ref, b3_ref, out_ref):
    w2 = w2_ref[...]
    w2m = w2[:, :D] + w2[:, D:2 * D]
    w2s = w2[:, 2 * D:]
    w3 = w3_ref[...]
    dnt = (((0,), (1,)), ((), ()))
    dn = (((1,), (1,)), ((), ()))
    h2 = (lax.dot_general(amaxT_ref[...], w2m, dnt,
                          preferred_element_type=jnp.float32)
          + lax.dot_general(asumT_ref[...], w2s, dnt,
                            preferred_element_type=jnp.float32)
          + b2_ref[...])
    h1 = (lax.dot_general(xdst_ref[...], w1_ref[...], dn,
                          preferred_element_type=jnp.float32)
          + b1_ref[...])
    h2 = jnp.maximum(h2, 0.0)
    h1 = jnp.maximum(h1, 0.0)
    o = (lax.dot_general(h2, w3[:, :H], dn,
                         preferred_element_type=jnp.float32)
         + lax.dot_general(h1, w3[:, H:], dn,
                           preferred_element_type=jnp.float32)
         + b3_ref[...])
    out_ref[...] = jnp.maximum(o, 0.0)


def _tc_mlp(amaxT, asumT, x_dst, fc1_w, fc1_b, fc2_w, fc2_b, fc3_w, fc3_b):
    return pl.pallas_call(
        _tc_mlp_body,
        out_shape=jax.ShapeDtypeStruct((N, H), jnp.float32),
    )(amaxT, asumT, x_dst, fc1_w, fc1_b.reshape(1, H), fc2_w,
      fc2_b.reshape(1, H), fc3_w, fc3_b.reshape(1, H))


@jax.jit
def kernel(x_src, x_dst, edge_index, fc1_w, fc1_b, fc2_w, fc2_b,
           fc3_w, fc3_b):
    src = edge_index[0]
    dst = edge_index[1]
    x_srcT_flat = x_src.T.reshape(-1)
    amaxT_flat, asumT_flat = _sc_aggregate(x_srcT_flat, src, dst)
    amaxT = amaxT_flat.reshape(D, N)
    asumT = asumT_flat.reshape(D, N)
    return _tc_mlp(amaxT, asumT, x_dst, fc1_w, fc1_b, fc2_w, fc2_b,
                   fc3_w, fc3_b)


# speedup vs baseline: 8.4201x; 2.0329x over previous
import jax
import jax.numpy as jnp
from jax import lax
from jax.experimental import pallas as pl
from jax.experimental.pallas import tpu as pltpu
from jax.experimental.pallas import tpu_sc as plsc

N = 10000
E = 320000
D = 128
H = 128

NW = 32
FPT = D // NW
CH = 2000
NCHUNK = E // CH
NEG = float("-inf")
LANES = 16


def _sc_agg_body(xsrcT_hbm, src_hbm, dst_hbm, amaxT_hbm, asumT_hbm,
                 dst_buf, src_buf, xslice, amax, asum, dsem, ssem):
    wid = lax.axis_index("s") * 2 + lax.axis_index("c")
    base = wid * (FPT * N)

    pltpu.sync_copy(xsrcT_hbm.at[pl.ds(base, FPT * N)], xslice)

    ninf = jnp.full((LANES,), NEG, jnp.float32)
    zero = jnp.zeros((LANES,), jnp.float32)

    def _init(i, _):
        s = pl.ds(i * LANES, LANES)
        amax[s] = ninf
        asum[s] = zero
        return 0
    lax.fori_loop(0, (FPT * N) // LANES, _init, 0)

    lane = lax.iota(jnp.int32, LANES)

    def _start_load(c, b):
        eb = c * CH
        pltpu.async_copy(dst_hbm.at[pl.ds(eb, CH)],
                         dst_buf.at[pl.ds(b * CH, CH)], dsem.at[b])
        pltpu.async_copy(src_hbm.at[pl.ds(eb, CH)],
                         src_buf.at[pl.ds(b * CH, CH)], ssem.at[b])

    def _wait_load(c, b):
        eb = c * CH
        pltpu.make_async_copy(dst_hbm.at[pl.ds(eb, CH)],
                              dst_buf.at[pl.ds(b * CH, CH)],
                              dsem.at[b]).wait()
        pltpu.make_async_copy(src_hbm.at[pl.ds(eb, CH)],
                              src_buf.at[pl.ds(b * CH, CH)],
                              ssem.at[b]).wait()

    _start_load(0, 0)

    def _chunk(c, _):
        b = lax.rem(c, 2)
        _wait_load(c, b)

        @pl.when(c + 1 < NCHUNK)
        def _():
            _start_load(c + 1, 1 - b)

        def _vreg(i, _):
            s = pl.ds(b * CH + i * LANES, LANES)
            dv = dst_buf[s]
            sv = src_buf[s]
            vs, aidxs = [], []
            need = None
            for f in range(FPT):
                gidx = sv + f * N
                aidx = dv + f * N
                v = plsc.load_gather(xslice, [gidx])
                plsc.addupdate_scatter(asum, [aidx], v)
                a = plsc.load_gather(amax, [aidx])
                plsc.store_scatter(amax, [aidx], jnp.maximum(a, v))
                a2 = plsc.load_gather(amax, [aidx])
                nf = a2 < v
                need = nf if need is None else (need | nf)
                vs.append(v)
                aidxs.append(aidx)

            def _fbody(carry):
                acc = None
                for f in range(FPT):
                    a2 = plsc.load_gather(amax, [aidxs[f]])
                    nf = a2 < vs[f]
                    eidx = jnp.where(nf, aidxs[f], FPT * N + lane)
                    plsc.store_scatter(amax, [eidx], jnp.maximum(a2, vs[f]))
                    acc = nf if acc is None else (acc | nf)
                return (jnp.any(acc), 0)
            lax.while_loop(lambda t: t[0], _fbody, (jnp.any(need), 0))
            return 0
        lax.fori_loop(0, CH // LANES, _vreg, 0)
        return 0
    lax.fori_loop(0, NCHUNK, _chunk, 0)

    def _fin(i, _):
        s = pl.ds(i * LANES, LANES)
        v = amax[s]
        amax[s] = jnp.where(v > NEG, v, 0.0)
        return 0
    lax.fori_loop(0, (FPT * N) // LANES, _fin, 0)

    pltpu.sync_copy(amax.at[pl.ds(0, FPT * N)],
                    amaxT_hbm.at[pl.ds(base, FPT * N)])
    pltpu.sync_copy(asum, asumT_hbm.at[pl.ds(base, FPT * N)])


def _sc_aggregate(x_srcT_flat, src, dst):
    mesh = plsc.VectorSubcoreMesh(core_axis_name="c", subcore_axis_name="s")
    f = pl.kernel(
        _sc_agg_body,
        out_type=[
            jax.ShapeDtypeStruct((D * N,), jnp.float32),
            jax.ShapeDtypeStruct((D * N,), jnp.float32),
        ],
        mesh=mesh,
        compiler_params=pltpu.CompilerParams(needs_layout_passes=False),
        scratch_types=[
            pltpu.VMEM((2 * CH,), jnp.int32),
            pltpu.VMEM((2 * CH,), jnp.int32),
            pltpu.VMEM((FPT * N,), jnp.float32),
            pltpu.VMEM((FPT * N + LANES,), jnp.float32),
            pltpu.VMEM((FPT * N,), jnp.float32),
            pltpu.SemaphoreType.DMA((2,)),
            pltpu.SemaphoreType.DMA((2,)),
        ],
    )
    return f(x_srcT_flat, src, dst)


def _tc_mlp_body(amaxT_ref, asumT_ref, xdst_ref, w1_ref, b1_ref, w2_ref,
                 b2_ref, w3_ref, b3_ref, out_ref):
    w2 = w2_ref[...]
    w2m = w2[:, :D] + w2[:, D:2 * D]
    w2s = w2[:, 2 * D:]
    w3 = w3_ref[...]
    dnt = (((0,), (1,)), ((), ()))
    dn = (((1,), (1,)), ((), ()))
    h2 = (lax.dot_general(amaxT_ref[...], w2m, dnt,
                          preferred_element_type=jnp.float32)
          + lax.dot_general(asumT_ref[...], w2s, dnt,
                            preferred_element_type=jnp.float32)
          + b2_ref[...])
    h1 = (lax.dot_general(xdst_ref[...], w1_ref[...], dn,
                          preferred_element_type=jnp.float32)
          + b1_ref[...])
    h2 = jnp.maximum(h2, 0.0)
    h1 = jnp.maximum(h1, 0.0)
    o = (lax.dot_general(h2, w3[:, :H], dn,
                         preferred_element_type=jnp.float32)
         + lax.dot_general(h1, w3[:, H:], dn,
                           preferred_element_type=jnp.float32)
         + b3_ref[...])
    out_ref[...] = jnp.maximum(o, 0.0)


def _tc_mlp(amaxT, asumT, x_dst, fc1_w, fc1_b, fc2_w, fc2_b, fc3_w, fc3_b):
    return pl.pallas_call(
        _tc_mlp_body,
        out_shape=jax.ShapeDtypeStruct((N, H), jnp.float32),
    )(amaxT, asumT, x_dst, fc1_w, fc1_b.reshape(1, H), fc2_w,
      fc2_b.reshape(1, H), fc3_w, fc3_b.reshape(1, H))


@jax.jit
def kernel(x_src, x_dst, edge_index, fc1_w, fc1_b, fc2_w, fc2_b,
           fc3_w, fc3_b):
    src = edge_index[0]
    dst = edge_index[1]
    x_srcT_flat = x_src.T.reshape(-1)
    amaxT_flat, asumT_flat = _sc_aggregate(x_srcT_flat, src, dst)
    amaxT = amaxT_flat.reshape(D, N)
    asumT = asumT_flat.reshape(D, N)
    return _tc_mlp(amaxT, asumT, x_dst, fc1_w, fc1_b, fc2_w, fc2_b,
                   fc3_w, fc3_b)


# speedup vs baseline: 14.3923x; 1.7093x over previous
import jax
import jax.numpy as jnp
from jax import lax
from jax.experimental import pallas as pl
from jax.experimental.pallas import tpu as pltpu
from jax.experimental.pallas import tpu_sc as plsc

N = 10000
E = 320000
D = 128
H = 128

NW = 32
FPT = D // NW
CH = 1600
NCHUNK = E // CH
NEG = float("-inf")
LANES = 16
HASHM = 2047


def _sc_agg_body(xsrcT_hbm, src_hbm, dst_hbm, amaxT_hbm, asumT_hbm,
                 dst_buf, src_buf, xslice, amax, asum, dtab, dsem, ssem):
    wid = lax.axis_index("s") * 2 + lax.axis_index("c")
    base = wid * (FPT * N)

    pltpu.sync_copy(xsrcT_hbm.at[pl.ds(base, FPT * N)], xslice)

    ninf = jnp.full((LANES,), NEG, jnp.float32)
    zero = jnp.zeros((LANES,), jnp.float32)

    def _init(i, _):
        s = pl.ds(i * LANES, LANES)
        amax[s] = ninf
        asum[s] = zero
        return 0
    lax.fori_loop(0, (FPT * N) // LANES, _init, 0)

    lane = lax.iota(jnp.int32, LANES)

    def _start_load(c, b):
        eb = c * CH
        pltpu.async_copy(dst_hbm.at[pl.ds(eb, CH)],
                         dst_buf.at[pl.ds(b * CH, CH)], dsem.at[b])
        pltpu.async_copy(src_hbm.at[pl.ds(eb, CH)],
                         src_buf.at[pl.ds(b * CH, CH)], ssem.at[b])

    def _wait_load(c, b):
        eb = c * CH
        pltpu.make_async_copy(dst_hbm.at[pl.ds(eb, CH)],
                              dst_buf.at[pl.ds(b * CH, CH)],
                              dsem.at[b]).wait()
        pltpu.make_async_copy(src_hbm.at[pl.ds(eb, CH)],
                              src_buf.at[pl.ds(b * CH, CH)],
                              ssem.at[b]).wait()

    _start_load(0, 0)

    def _chunk(c, _):
        b = lax.rem(c, 2)
        _wait_load(c, b)

        @pl.when(c + 1 < NCHUNK)
        def _():
            _start_load(c + 1, 1 - b)

        def _vreg(i, _):
            s = pl.ds(b * CH + i * LANES, LANES)
            dv = dst_buf[s]
            sv = src_buf[s]

            hidx = jnp.bitwise_and(dv, HASHM)
            plsc.store_scatter(dtab, [hidx], lane)
            dup = plsc.load_gather(dtab, [hidx]) != lane

            vs, aidxs = [], []
            for f in range(FPT):
                gidx = sv + f * N
                aidx = dv + f * N
                vs.append(plsc.load_gather(xslice, [gidx]))
                aidxs.append(aidx)
            ms = [jnp.maximum(plsc.load_gather(amax, [aidxs[f]]), vs[f])
                  for f in range(FPT)]
            for f in range(FPT):
                plsc.addupdate_scatter(asum, [aidxs[f]], vs[f])
                plsc.store_scatter(amax, [aidxs[f]], ms[f])

            def _fbody(carry):
                acc = None
                for f in range(FPT):
                    a2 = plsc.load_gather(amax, [aidxs[f]])
                    nf = a2 < vs[f]
                    eidx = jnp.where(nf, aidxs[f], FPT * N + lane)
                    plsc.store_scatter(amax, [eidx], jnp.maximum(a2, vs[f]))
                    acc = nf if acc is None else (acc | nf)
                return (jnp.any(acc), 0)
            lax.while_loop(lambda t: t[0], _fbody, (jnp.any(dup), 0))
            return 0
        lax.fori_loop(0, CH // LANES, _vreg, 0)
        return 0
    lax.fori_loop(0, NCHUNK, _chunk, 0)

    def _fin(i, _):
        s = pl.ds(i * LANES, LANES)
        v = amax[s]
        amax[s] = jnp.where(v > NEG, v, 0.0)
        return 0
    lax.fori_loop(0, (FPT * N) // LANES, _fin, 0)

    pltpu.sync_copy(amax.at[pl.ds(0, FPT * N)],
                    amaxT_hbm.at[pl.ds(base, FPT * N)])
    pltpu.sync_copy(asum, asumT_hbm.at[pl.ds(base, FPT * N)])


def _sc_aggregate(x_srcT_flat, src, dst):
    mesh = plsc.VectorSubcoreMesh(core_axis_name="c", subcore_axis_name="s")
    f = pl.kernel(
        _sc_agg_body,
        out_type=[
            jax.ShapeDtypeStruct((D * N,), jnp.float32),
            jax.ShapeDtypeStruct((D * N,), jnp.float32),
        ],
        mesh=mesh,
        compiler_params=pltpu.CompilerParams(needs_layout_passes=False),
        scratch_types=[
            pltpu.VMEM((2 * CH,), jnp.int32),
            pltpu.VMEM((2 * CH,), jnp.int32),
            pltpu.VMEM((FPT * N,), jnp.float32),
            pltpu.VMEM((FPT * N + LANES,), jnp.float32),
            pltpu.VMEM((FPT * N,), jnp.float32),
            pltpu.VMEM((HASHM + 1,), jnp.int32),
            pltpu.SemaphoreType.DMA((2,)),
            pltpu.SemaphoreType.DMA((2,)),
        ],
    )
    return f(x_srcT_flat, src, dst)


def _tc_mlp_body(amaxT_ref, asumT_ref, xdst_ref, w1_ref, b1_ref, w2_ref,
                 b2_ref, w3_ref, b3_ref, out_ref):
    w2 = w2_ref[...]
    w2m = w2[:, :D] + w2[:, D:2 * D]
    w2s = w2[:, 2 * D:]
    w3 = w3_ref[...]
    dnt = (((0,), (1,)), ((), ()))
    dn = (((1,), (1,)), ((), ()))
    h2 = (lax.dot_general(amaxT_ref[...], w2m, dnt,
                          preferred_element_type=jnp.float32)
          + lax.dot_general(asumT_ref[...], w2s, dnt,
                            preferred_element_type=jnp.float32)
          + b2_ref[...])
    h1 = (lax.dot_general(xdst_ref[...], w1_ref[...], dn,
                          preferred_element_type=jnp.float32)
          + b1_ref[...])
    h2 = jnp.maximum(h2, 0.0)
    h1 = jnp.maximum(h1, 0.0)
    o = (lax.dot_general(h2, w3[:, :H], dn,
                         preferred_element_type=jnp.float32)
         + lax.dot_general(h1, w3[:, H:], dn,
                           preferred_element_type=jnp.float32)
         + b3_ref[...])
    out_ref[...] = jnp.maximum(o, 0.0)


def _tc_mlp(amaxT, asumT, x_dst, fc1_w, fc1_b, fc2_w, fc2_b, fc3_w, fc3_b):
    return pl.pallas_call(
        _tc_mlp_body,
        out_shape=jax.ShapeDtypeStruct((N, H), jnp.float32),
    )(amaxT, asumT, x_dst, fc1_w, fc1_b.reshape(1, H), fc2_w,
      fc2_b.reshape(1, H), fc3_w, fc3_b.reshape(1, H))


@jax.jit
def kernel(x_src, x_dst, edge_index, fc1_w, fc1_b, fc2_w, fc2_b,
           fc3_w, fc3_b):
    src = edge_index[0]
    dst = edge_index[1]
    x_srcT_flat = x_src.T.reshape(-1)
    amaxT_flat, asumT_flat = _sc_aggregate(x_srcT_flat, src, dst)
    amaxT = amaxT_flat.reshape(D, N)
    asumT = asumT_flat.reshape(D, N)
    return _tc_mlp(amaxT, asumT, x_dst, fc1_w, fc1_b, fc2_w, fc2_b,
                   fc3_w, fc3_b)


# speedup vs baseline: 17.2533x; 1.1988x over previous
import jax
import jax.numpy as jnp
from jax import lax
from jax.experimental import pallas as pl
from jax.experimental.pallas import tpu as pltpu
from jax.experimental.pallas import tpu_sc as plsc

N = 10000
E = 320000
D = 128
H = 128

NW = 32
FPT = D // NW
CH = 1600
NCHUNK = E // CH
NEG = float("-inf")
LANES = 16
HASHM = 2047
UNROLL = 2


def _sc_agg_body(xsrcT_hbm, src_hbm, dst_hbm, amaxT_hbm, asumT_hbm,
                 dst_buf, src_buf, xslice, amax, asum, dtab, dsem, ssem):
    wid = lax.axis_index("s") * 2 + lax.axis_index("c")
    base = wid * (FPT * N)

    pltpu.sync_copy(xsrcT_hbm.at[pl.ds(base, FPT * N)], xslice)

    ninf = jnp.full((LANES,), NEG, jnp.float32)
    zero = jnp.zeros((LANES,), jnp.float32)

    def _init(i, _):
        s = pl.ds(i * LANES, LANES)
        amax[s] = ninf
        asum[s] = zero
        return 0
    lax.fori_loop(0, (FPT * N) // LANES, _init, 0)

    lane = lax.iota(jnp.int32, LANES)

    def _start_load(c, b):
        eb = c * CH
        pltpu.async_copy(dst_hbm.at[pl.ds(eb, CH)],
                         dst_buf.at[pl.ds(b * CH, CH)], dsem.at[b])
        pltpu.async_copy(src_hbm.at[pl.ds(eb, CH)],
                         src_buf.at[pl.ds(b * CH, CH)], ssem.at[b])

    def _wait_load(c, b):
        eb = c * CH
        pltpu.make_async_copy(dst_hbm.at[pl.ds(eb, CH)],
                              dst_buf.at[pl.ds(b * CH, CH)],
                              dsem.at[b]).wait()
        pltpu.make_async_copy(src_hbm.at[pl.ds(eb, CH)],
                              src_buf.at[pl.ds(b * CH, CH)],
                              ssem.at[b]).wait()

    _start_load(0, 0)

    def _chunk(c, _):
        b = lax.rem(c, 2)
        _wait_load(c, b)

        @pl.when(c + 1 < NCHUNK)
        def _():
            _start_load(c + 1, 1 - b)

        def _vreg(i2, _):
            vs, aidxs, hidxs = [], [], []
            for u in range(UNROLL):
                s = pl.ds(b * CH + (i2 * UNROLL + u) * LANES, LANES)
                dv = dst_buf[s]
                sv = src_buf[s]
                hidxs.append(jnp.bitwise_and(dv, HASHM))
                for f in range(FPT):
                    vs.append(plsc.load_gather(xslice, [sv + f * N]))
                    aidxs.append(dv + f * N)

            for u in range(UNROLL):
                plsc.store_scatter(dtab, [hidxs[u]], lane + u * LANES)
            dup = None
            for u in range(UNROLL):
                du = plsc.load_gather(dtab, [hidxs[u]]) != lane + u * LANES
                dup = du if dup is None else (dup | du)
            ms = [jnp.maximum(plsc.load_gather(amax, [aidxs[k]]), vs[k])
                  for k in range(UNROLL * FPT)]
            for k in range(UNROLL * FPT):
                plsc.addupdate_scatter(asum, [aidxs[k]], vs[k])
                plsc.store_scatter(amax, [aidxs[k]], ms[k])

            def _fbody(carry):
                acc = None
                for k in range(UNROLL * FPT):
                    a2 = plsc.load_gather(amax, [aidxs[k]])
                    nf = a2 < vs[k]
                    eidx = jnp.where(nf, aidxs[k], FPT * N + lane)
                    plsc.store_scatter(amax, [eidx], jnp.maximum(a2, vs[k]))
                    acc = nf if acc is None else (acc | nf)
                return (jnp.any(acc), 0)
            lax.while_loop(lambda t: t[0], _fbody, (jnp.any(dup), 0))
            return 0
        lax.fori_loop(0, CH // (LANES * UNROLL), _vreg, 0)
        return 0
    lax.fori_loop(0, NCHUNK, _chunk, 0)

    def _fin(i, _):
        s = pl.ds(i * LANES, LANES)
        v = amax[s]
        amax[s] = jnp.where(v > NEG, v, 0.0)
        return 0
    lax.fori_loop(0, (FPT * N) // LANES, _fin, 0)

    pltpu.sync_copy(amax.at[pl.ds(0, FPT * N)],
                    amaxT_hbm.at[pl.ds(base, FPT * N)])
    pltpu.sync_copy(asum, asumT_hbm.at[pl.ds(base, FPT * N)])


def _sc_aggregate(x_srcT_flat, src, dst):
    mesh = plsc.VectorSubcoreMesh(core_axis_name="c", subcore_axis_name="s")
    f = pl.kernel(
        _sc_agg_body,
        out_type=[
            jax.ShapeDtypeStruct((D * N,), jnp.float32),
            jax.ShapeDtypeStruct((D * N,), jnp.float32),
        ],
        mesh=mesh,
        compiler_params=pltpu.CompilerParams(needs_layout_passes=False),
        scratch_types=[
            pltpu.VMEM((2 * CH,), jnp.int32),
            pltpu.VMEM((2 * CH,), jnp.int32),
            pltpu.VMEM((FPT * N,), jnp.float32),
            pltpu.VMEM((FPT * N + LANES,), jnp.float32),
            pltpu.VMEM((FPT * N,), jnp.float32),
            pltpu.VMEM((HASHM + 1,), jnp.int32),
            pltpu.SemaphoreType.DMA((2,)),
            pltpu.SemaphoreType.DMA((2,)),
        ],
    )
    return f(x_srcT_flat, src, dst)


def _tc_mlp_body(amaxT_ref, asumT_ref, xdst_ref, w1_ref, b1_ref, w2_ref,
                 b2_ref, w3_ref, b3_ref, out_ref):
    w2 = w2_ref[...]
    w2m = w2[:, :D] + w2[:, D:2 * D]
    w2s = w2[:, 2 * D:]
    w3 = w3_ref[...]
    dnt = (((0,), (1,)), ((), ()))
    dn = (((1,), (1,)), ((), ()))
    h2 = (lax.dot_general(amaxT_ref[...], w2m, dnt,
                          preferred_element_type=jnp.float32)
          + lax.dot_general(asumT_ref[...], w2s, dnt,
                            preferred_element_type=jnp.float32)
          + b2_ref[...])
    h1 = (lax.dot_general(xdst_ref[...], w1_ref[...], dn,
                          preferred_element_type=jnp.float32)
          + b1_ref[...])
    h2 = jnp.maximum(h2, 0.0)
    h1 = jnp.maximum(h1, 0.0)
    o = (lax.dot_general(h2, w3[:, :H], dn,
                         preferred_element_type=jnp.float32)
         + lax.dot_general(h1, w3[:, H:], dn,
                           preferred_element_type=jnp.float32)
         + b3_ref[...])
    out_ref[...] = jnp.maximum(o, 0.0)


def _tc_mlp(amaxT, asumT, x_dst, fc1_w, fc1_b, fc2_w, fc2_b, fc3_w, fc3_b):
    return pl.pallas_call(
        _tc_mlp_body,
        out_shape=jax.ShapeDtypeStruct((N, H), jnp.float32),
    )(amaxT, asumT, x_dst, fc1_w, fc1_b.reshape(1, H), fc2_w,
      fc2_b.reshape(1, H), fc3_w, fc3_b.reshape(1, H))


@jax.jit
def kernel(x_src, x_dst, edge_index, fc1_w, fc1_b, fc2_w, fc2_b,
           fc3_w, fc3_b):
    src = edge_index[0]
    dst = edge_index[1]
    x_srcT_flat = x_src.T.reshape(-1)
    amaxT_flat, asumT_flat = _sc_aggregate(x_srcT_flat, src, dst)
    amaxT = amaxT_flat.reshape(D, N)
    asumT = asumT_flat.reshape(D, N)
    return _tc_mlp(amaxT, asumT, x_dst, fc1_w, fc1_b, fc2_w, fc2_b,
                   fc3_w, fc3_b)


# speedup vs baseline: 19.2424x; 1.1153x over previous
import jax
import jax.numpy as jnp
from jax import lax
from jax.experimental import pallas as pl
from jax.experimental.pallas import tpu as pltpu
from jax.experimental.pallas import tpu_sc as plsc

N = 10000
E = 320000
D = 128
H = 128

NW = 32
FPT = D // NW
CH = 1600
NCHUNK = E // CH
NEG = float("-inf")
LANES = 16
HASHM = 4095
UNROLL = 2


def _sc_agg_body(xsrcT_hbm, src_hbm, dst_hbm, amaxT_hbm, asumT_hbm,
                 dst_buf, src_buf, xslice, amax, asum, dtab, dsem, ssem):
    wid = lax.axis_index("s") * 2 + lax.axis_index("c")
    base = wid * (FPT * N)

    pltpu.sync_copy(xsrcT_hbm.at[pl.ds(base, FPT * N)], xslice)

    ninf = jnp.full((LANES,), NEG, jnp.float32)
    zero = jnp.zeros((LANES,), jnp.float32)

    def _init(i, _):
        s = pl.ds(i * LANES, LANES)
        amax[s] = ninf
        asum[s] = zero
        return 0
    lax.fori_loop(0, (FPT * N) // LANES, _init, 0)

    lane = lax.iota(jnp.int32, LANES)

    def _start_load(c, b):
        eb = c * CH
        pltpu.async_copy(dst_hbm.at[pl.ds(eb, CH)],
                         dst_buf.at[pl.ds(b * CH, CH)], dsem.at[b])
        pltpu.async_copy(src_hbm.at[pl.ds(eb, CH)],
                         src_buf.at[pl.ds(b * CH, CH)], ssem.at[b])

    def _wait_load(c, b):
        eb = c * CH
        pltpu.make_async_copy(dst_hbm.at[pl.ds(eb, CH)],
                              dst_buf.at[pl.ds(b * CH, CH)],
                              dsem.at[b]).wait()
        pltpu.make_async_copy(src_hbm.at[pl.ds(eb, CH)],
                              src_buf.at[pl.ds(b * CH, CH)],
                              ssem.at[b]).wait()

    _start_load(0, 0)

    def _chunk(c, _):
        b = lax.rem(c, 2)
        _wait_load(c, b)

        @pl.when(c + 1 < NCHUNK)
        def _():
            _start_load(c + 1, 1 - b)

        def _vreg(i2, _):
            vs, aidxs, hidxs = [], [], []
            for u in range(UNROLL):
                s = pl.ds(b * CH + (i2 * UNROLL + u) * LANES, LANES)
                dv = dst_buf[s]
                sv = src_buf[s]
                hidxs.append(jnp.bitwise_and(dv, HASHM))
                for f in range(FPT):
                    vs.append(plsc.load_gather(xslice, [sv + f * N]))
                    aidxs.append(dv + f * N)

            for u in range(UNROLL):
                plsc.store_scatter(dtab, [hidxs[u]], lane + u * LANES)
            dup = None
            for u in range(UNROLL):
                du = plsc.load_gather(dtab, [hidxs[u]]) != lane + u * LANES
                dup = du if dup is None else (dup | du)
            ms = [jnp.maximum(plsc.load_gather(amax, [aidxs[k]]), vs[k])
                  for k in range(UNROLL * FPT)]
            for k in range(UNROLL * FPT):
                plsc.addupdate_scatter(asum, [aidxs[k]], vs[k])
                plsc.store_scatter(amax, [aidxs[k]], ms[k])

            def _fbody(carry):
                acc = None
                for k in range(UNROLL * FPT):
                    a2 = plsc.load_gather(amax, [aidxs[k]])
                    nf = a2 < vs[k]
                    eidx = jnp.where(nf, aidxs[k], FPT * N + lane)
                    plsc.store_scatter(amax, [eidx], jnp.maximum(a2, vs[k]))
                    acc = nf if acc is None else (acc | nf)
                return (jnp.any(acc), 0)
            lax.while_loop(lambda t: t[0], _fbody, (jnp.any(dup), 0))
            return 0
        lax.fori_loop(0, CH // (LANES * UNROLL), _vreg, 0)
        return 0
    lax.fori_loop(0, NCHUNK, _chunk, 0)

    def _fin(i, _):
        s = pl.ds(i * LANES, LANES)
        v = amax[s]
        amax[s] = jnp.where(v > NEG, v, 0.0)
        return 0
    lax.fori_loop(0, (FPT * N) // LANES, _fin, 0)

    pltpu.sync_copy(amax.at[pl.ds(0, FPT * N)],
                    amaxT_hbm.at[pl.ds(base, FPT * N)])
    pltpu.sync_copy(asum, asumT_hbm.at[pl.ds(base, FPT * N)])


def _sc_aggregate(x_srcT_flat, src, dst):
    mesh = plsc.VectorSubcoreMesh(core_axis_name="c", subcore_axis_name="s")
    f = pl.kernel(
        _sc_agg_body,
        out_type=[
            jax.ShapeDtypeStruct((D * N,), jnp.float32),
            jax.ShapeDtypeStruct((D * N,), jnp.float32),
        ],
        mesh=mesh,
        compiler_params=pltpu.CompilerParams(needs_layout_passes=False),
        scratch_types=[
            pltpu.VMEM((2 * CH,), jnp.int32),
            pltpu.VMEM((2 * CH,), jnp.int32),
            pltpu.VMEM((FPT * N,), jnp.float32),
            pltpu.VMEM((FPT * N + LANES,), jnp.float32),
            pltpu.VMEM((FPT * N,), jnp.float32),
            pltpu.VMEM((HASHM + 1,), jnp.int32),
            pltpu.SemaphoreType.DMA((2,)),
            pltpu.SemaphoreType.DMA((2,)),
        ],
    )
    return f(x_srcT_flat, src, dst)


def _tc_mlp_body(amaxT_ref, asumT_ref, xdst_ref, w1_ref, b1_ref, w2_ref,
                 b2_ref, w3_ref, b3_ref, out_ref):
    w2 = w2_ref[...]
    w2m = w2[:, :D] + w2[:, D:2 * D]
    w2s = w2[:, 2 * D:]
    w3 = w3_ref[...]
    dnt = (((0,), (1,)), ((), ()))
    dn = (((1,), (1,)), ((), ()))
    h2 = (lax.dot_general(amaxT_ref[...], w2m, dnt,
                          preferred_element_type=jnp.float32)
          + lax.dot_general(asumT_ref[...], w2s, dnt,
                            preferred_element_type=jnp.float32)
          + b2_ref[...])
    h1 = (lax.dot_general(xdst_ref[...], w1_ref[...], dn,
                          preferred_element_type=jnp.float32)
          + b1_ref[...])
    h2 = jnp.maximum(h2, 0.0)
    h1 = jnp.maximum(h1, 0.0)
    o = (lax.dot_general(h2, w3[:, :H], dn,
                         preferred_element_type=jnp.float32)
         + lax.dot_general(h1, w3[:, H:], dn,
                           preferred_element_type=jnp.float32)
         + b3_ref[...])
    out_ref[...] = jnp.maximum(o, 0.0)


def _tc_mlp(amaxT, asumT, x_dst, fc1_w, fc1_b, fc2_w, fc2_b, fc3_w, fc3_b):
    return pl.pallas_call(
        _tc_mlp_body,
        out_shape=jax.ShapeDtypeStruct((N, H), jnp.float32),
    )(amaxT, asumT, x_dst, fc1_w, fc1_b.reshape(1, H), fc2_w,
      fc2_b.reshape(1, H), fc3_w, fc3_b.reshape(1, H))


@jax.jit
def kernel(x_src, x_dst, edge_index, fc1_w, fc1_b, fc2_w, fc2_b,
           fc3_w, fc3_b):
    src = edge_index[0]
    dst = edge_index[1]
    x_srcT_flat = x_src.T.reshape(-1)
    amaxT_flat, asumT_flat = _sc_aggregate(x_srcT_flat, src, dst)
    amaxT = amaxT_flat.reshape(D, N)
    asumT = asumT_flat.reshape(D, N)
    return _tc_mlp(amaxT, asumT, x_dst, fc1_w, fc1_b, fc2_w, fc2_b,
                   fc3_w, fc3_b)
